# Initial kernel scaffold; baseline (speedup 1.0000x reference)
import jax
import jax.numpy as jnp
from jax.experimental import pallas as pl


def kernel(decoder_logits, bucket_amplitude, perm_1d, raw_temperature):
    raise NotImplementedError("write your pallas kernel here")


# speedup vs baseline: 94.0748x; 94.0748x over previous
import math

import jax
import jax.numpy as jnp
from jax.experimental import pallas as pl
from jax.experimental.pallas import tpu as pltpu

MIN_TEMP = 0.05


def _head_kernel(inv_t_ref, logits_ref, amp_ref,
                 probs_ref, colsum_ref, doubt_ref, support_ref):
    inv_t = inv_t_ref[0]
    x = logits_ref[0] * inv_t
    m = jnp.max(x, axis=-1, keepdims=True)
    e = jnp.exp(x - m)
    s = jnp.sum(e, axis=-1, keepdims=True)
    inv_s = 1.0 / s
    p = e * inv_s
    probs_ref[0] = p
    amp = amp_ref[0]
    colsum_ref[0] = jax.lax.dot_general(
        amp, p, (((1,), (0,)), ((), ())),
        preferred_element_type=jnp.float32)
    px = jnp.sum(e * x, axis=-1, keepdims=True) * inv_s
    ent = m[:, 0] + jnp.log(s[:, 0]) - px[:, 0]
    n = logits_ref.shape[-1]
    doubt_ref[0, 0] = ent * (1.0 / math.log(float(n)))
    support_ref[0, 0] = jnp.exp(ent)


def kernel(decoder_logits, bucket_amplitude, perm_1d, raw_temperature):
    B, C, N = decoder_logits.shape
    temperature = jnp.asarray(MIN_TEMP, decoder_logits.dtype) + jax.nn.softplus(
        raw_temperature).astype(decoder_logits.dtype)
    inv_t = (1.0 / temperature).reshape(1)
    amp3 = bucket_amplitude.reshape(B, 1, C)

    grid_spec = pltpu.PrefetchScalarGridSpec(
        num_scalar_prefetch=1,
        grid=(B,),
        in_specs=[
            pl.BlockSpec((1, C, N), lambda b, s: (b, 0, 0)),
            pl.BlockSpec((1, 1, C), lambda b, s: (b, 0, 0)),
        ],
        out_specs=[
            pl.BlockSpec((1, C, N), lambda b, s: (b, 0, 0)),
            pl.BlockSpec((1, 1, N), lambda b, s: (b, 0, 0)),
            pl.BlockSpec((1, 1, C), lambda b, s: (b, 0, 0)),
            pl.BlockSpec((1, 1, C), lambda b, s: (b, 0, 0)),
        ],
    )
    probs, colsum, doubt, support = pl.pallas_call(
        _head_kernel,
        grid_spec=grid_spec,
        out_shape=[
            jax.ShapeDtypeStruct((B, C, N), decoder_logits.dtype),
            jax.ShapeDtypeStruct((B, 1, N), decoder_logits.dtype),
            jax.ShapeDtypeStruct((B, 1, C), decoder_logits.dtype),
            jax.ShapeDtypeStruct((B, 1, C), decoder_logits.dtype),
        ],
        compiler_params=pltpu.CompilerParams(
            dimension_semantics=("arbitrary",),
        ),
    )(inv_t, decoder_logits, amp3)

    colsum = colsum.reshape(B, N)
    idx = perm_1d.astype(jnp.int32)
    reconstruction = jnp.zeros((B, N), decoder_logits.dtype).at[:, idx].add(colsum)
    return (reconstruction, probs, doubt.reshape(B, C), support.reshape(B, C),
            temperature)
